# baseline (speedup 1.0000x reference)
import functools

import jax
import jax.numpy as jnp
from jax import lax
from jax.experimental import pallas as pl
from jax.experimental.pallas import tpu as pltpu
from jax.experimental.pallas import tpu_sc as plsc

N_NODES = 10000
N_EDGES = 320000
D = 128

NC = 2
NS = 16
NW = NC * NS
CHUNK = 128
NCHUNK = 80
SCHUNK = 20
EPW = NCHUNK * CHUNK
NPAD = 10240
RPT = NPAD // NS
VPC = CHUNK // 16


def _sc_aggregate(x, ei4):
    mesh = plsc.VectorSubcoreMesh(
        core_axis_name="c", subcore_axis_name="s", num_cores=NC, num_subcores=NS
    )

    @functools.partial(
        pl.kernel,
        out_type=[
            jax.ShapeDtypeStruct((NC, NPAD, D), jnp.float32),
            jax.ShapeDtypeStruct((NC, NS, NPAD), jnp.float32),
        ],
        mesh=mesh,
        compiler_params=pltpu.CompilerParams(needs_layout_passes=False),
        scratch_types=[
            pltpu.VMEM((SCHUNK, 2, CHUNK), jnp.int32),
            pltpu.VMEM((CHUNK, D), jnp.float32),
            pltpu.VMEM((CHUNK, D), jnp.float32),
            pltpu.VMEM((NPAD,), jnp.float32),
            pltpu.VMEM_SHARED((NPAD, D), jnp.float32),
            pltpu.SemaphoreType.DMA,
            pltpu.SemaphoreType.DMA,
        ],
    )
    def agg(x_hbm, ei_hbm, part_hbm, hist_hbm,
            idx_v, rows_a, rows_b, hist_v, accum_sh, sem_g, sem_s):
        c = lax.axis_index("c")
        s = lax.axis_index("s")
        wid = c * NS + s

        zeros16 = jnp.zeros((16,), jnp.float32)

        def z_hist(i, _):
            hist_v[pl.ds(i * 16, 16)] = zeros16
            return 0
        lax.fori_loop(0, NPAD // 16, z_hist, 0)

        def z_rows(i, _):
            for v in range(D // 16):
                rows_a[i, pl.ds(v * 16, 16)] = zeros16
            return 0
        lax.fori_loop(0, CHUNK, z_rows, 0)

        for k in range(RPT // CHUNK):
            pltpu.sync_copy(rows_a, accum_sh.at[pl.ds(s * RPT + k * CHUNK, CHUNK)])
        plsc.subcore_barrier()

        def hist_update(sl):
            for v in range(VPC):
                dvec = idx_v[sl, 1, pl.ds(v * 16, 16)]
                occ, last = plsc.scan_count(dvec)
                cur = plsc.load_gather(hist_v, [dvec], mask=last)
                plsc.store_scatter(hist_v, [dvec],
                                   cur + occ.astype(jnp.float32), mask=last)

        def stage(st, _):
            pltpu.sync_copy(ei_hbm.at[wid, pl.ds(st * SCHUNK, SCHUNK)], idx_v)

            def pair(p, _):
                ja = 2 * p
                jb = 2 * p + 1
                ga = pltpu.async_copy(x_hbm.at[idx_v.at[ja, 0]], rows_a, sem_g)
                gb = pltpu.async_copy(x_hbm.at[idx_v.at[jb, 0]], rows_b, sem_g)
                ga.wait()
                gb.wait()
                sa = pltpu.async_copy(rows_a, accum_sh.at[idx_v.at[ja, 1]],
                                      sem_s, add=True)
                sb = pltpu.async_copy(rows_b, accum_sh.at[idx_v.at[jb, 1]],
                                      sem_s, add=True)
                hist_update(ja)
                hist_update(jb)
                sa.wait()
                sb.wait()
                return 0

            lax.fori_loop(0, SCHUNK // 2, pair, 0)
            return 0

        lax.fori_loop(0, NCHUNK // SCHUNK, stage, 0)
        plsc.subcore_barrier()

        pltpu.sync_copy(accum_sh.at[pl.ds(s * RPT, RPT)],
                        part_hbm.at[c, pl.ds(s * RPT, RPT)])
        pltpu.sync_copy(hist_v, hist_hbm.at[c, s])

    return agg(x, ei4)


def _tc_combine_body(p_ref, h_ref, x_ref, w_ref, b_ref, o_ref):
    agg = p_ref[0] + p_ref[1]
    deg = jnp.sum(h_ref[...].reshape(NW, -1), axis=0)
    deg = jnp.maximum(deg, 1.0)
    agg = agg / deg[:, None]
    o_ref[...] = (
        lax.dot_general(agg, w_ref[...], (((1,), (1,)), ((), ())),
                        preferred_element_type=jnp.float32)
        + lax.dot_general(x_ref[...], b_ref[...], (((1,), (1,)), ((), ())),
                          preferred_element_type=jnp.float32)
    )


def _tc_combine(part, deg, x_pad, W, B):
    BLK = 1024
    grid = (NPAD // BLK,)
    return pl.pallas_call(
        _tc_combine_body,
        grid=grid,
        in_specs=[
            pl.BlockSpec((NC, BLK, D), lambda i: (0, i, 0)),
            pl.BlockSpec((NC, NS, BLK), lambda i: (0, 0, i)),
            pl.BlockSpec((BLK, D), lambda i: (i, 0)),
            pl.BlockSpec((D, D), lambda i: (0, 0)),
            pl.BlockSpec((D, D), lambda i: (0, 0)),
        ],
        out_specs=pl.BlockSpec((BLK, D), lambda i: (i, 0)),
        out_shape=jax.ShapeDtypeStruct((NPAD, D), jnp.float32),
    )(part, deg, x_pad, W, B)


def kernel(x, edge_index, W, B):
    src = edge_index[0].astype(jnp.int32)
    dst = edge_index[1].astype(jnp.int32)
    pad = NW * EPW - N_EDGES
    src_p = jnp.concatenate([src, jnp.zeros((pad,), jnp.int32)])
    dst_p = jnp.concatenate([dst, jnp.full((pad,), N_NODES, jnp.int32)])
    src3 = src_p.reshape(NW, NCHUNK, CHUNK)
    dst3 = dst_p.reshape(NW, NCHUNK, CHUNK)
    ei4 = jnp.stack([src3, dst3], axis=2)

    part, deg = _sc_aggregate(x, ei4)
    x_pad = jnp.pad(x, ((0, NPAD - N_NODES), (0, 0)))
    out = _tc_combine(part, deg, x_pad, W, B)
    return out[:N_NODES]


# speedup vs baseline: 1.0636x; 1.0636x over previous
import functools

import jax
import jax.numpy as jnp
from jax import lax
from jax.experimental import pallas as pl
from jax.experimental.pallas import tpu as pltpu
from jax.experimental.pallas import tpu_sc as plsc

N_NODES = 10000
N_EDGES = 320000
D = 128

NC = 2
NS = 16
NW = NC * NS
CHUNK = 128
NCH0 = 110
NCH1 = 50
TOTC = NS * (NCH0 + NCH1)
SSTG = 10
E_PAD = TOTC * CHUNK
NPAD = 10240
RPT = NPAD // NS
VPC = CHUNK // 16


def _sc_aggregate(x, ei3):
    mesh = plsc.VectorSubcoreMesh(
        core_axis_name="c", subcore_axis_name="s", num_cores=NC, num_subcores=NS
    )

    @functools.partial(
        pl.kernel,
        out_type=[
            jax.ShapeDtypeStruct((NC, NPAD, D), jnp.float32),
            jax.ShapeDtypeStruct((NC, NS, NPAD), jnp.float32),
        ],
        mesh=mesh,
        compiler_params=pltpu.CompilerParams(needs_layout_passes=False),
        scratch_types=[
            pltpu.VMEM((SSTG, 2, CHUNK), jnp.int32),
            pltpu.VMEM((CHUNK, D), jnp.float32),
            pltpu.VMEM((NPAD,), jnp.float32),
            pltpu.VMEM_SHARED((NPAD, D), jnp.float32),
            pltpu.SemaphoreType.DMA,
            pltpu.SemaphoreType.DMA,
        ],
    )
    def agg(x_hbm, ei_hbm, part_hbm, hist_hbm,
            idx_v, rows_a, hist_v, accum_sh, sem_g, sem_s):
        c = lax.axis_index("c")
        s = lax.axis_index("s")
        wid = c * NS + s

        zeros16 = jnp.zeros((16,), jnp.float32)

        def z_hist(i, _):
            hist_v[pl.ds(i * 16, 16)] = zeros16
            return 0
        lax.fori_loop(0, NPAD // 16, z_hist, 0)

        def z_rows(i, _):
            for v in range(D // 16):
                rows_a[i, pl.ds(v * 16, 16)] = zeros16
            return 0
        lax.fori_loop(0, CHUNK, z_rows, 0)

        for k in range(RPT // CHUNK):
            pltpu.sync_copy(rows_a, accum_sh.at[pl.ds(s * RPT + k * CHUNK, CHUNK)])
        plsc.subcore_barrier()

        def hist_update(sl):
            for v in range(VPC):
                dvec = idx_v[sl, 1, pl.ds(v * 16, 16)]
                occ, last = plsc.scan_count(dvec)
                cur = plsc.load_gather(hist_v, [dvec], mask=last)
                plsc.store_scatter(hist_v, [dvec],
                                   cur + occ.astype(jnp.float32), mask=last)

        nst = jnp.where(c == 0, NCH0 // SSTG, NCH1 // SSTG)
        base = jnp.where(c == 0, s * NCH0, NS * NCH0 + s * NCH1)

        def stage(st, _):
            pltpu.sync_copy(ei_hbm.at[pl.ds(base + st * SSTG, SSTG)], idx_v)

            def sc_body(p, _):
                pltpu.sync_copy(x_hbm.at[idx_v.at[p, 0]], rows_a)
                pltpu.sync_copy(rows_a, accum_sh.at[idx_v.at[p, 1]], add=True)
                hist_update(p)
                return 0

            lax.fori_loop(0, SSTG, sc_body, 0)
            return 0

        lax.fori_loop(0, nst, stage, 0)
        plsc.subcore_barrier()

        pltpu.sync_copy(accum_sh.at[pl.ds(s * RPT, RPT)],
                        part_hbm.at[c, pl.ds(s * RPT, RPT)])
        pltpu.sync_copy(hist_v, hist_hbm.at[c, s])

    return agg(x, ei3)


def _tc_combine_body(p_ref, h_ref, x_ref, w_ref, b_ref, o_ref):
    agg = p_ref[0] + p_ref[1]
    deg = jnp.sum(h_ref[...].reshape(NW, -1), axis=0)
    deg = jnp.maximum(deg, 1.0)
    agg = agg / deg[:, None]
    o_ref[...] = (
        lax.dot_general(agg, w_ref[...], (((1,), (1,)), ((), ())),
                        preferred_element_type=jnp.float32)
        + lax.dot_general(x_ref[...], b_ref[...], (((1,), (1,)), ((), ())),
                          preferred_element_type=jnp.float32)
    )


def _tc_combine(part, deg, x_pad, W, B):
    BLK = 1024
    grid = (NPAD // BLK,)
    return pl.pallas_call(
        _tc_combine_body,
        grid=grid,
        in_specs=[
            pl.BlockSpec((NC, BLK, D), lambda i: (0, i, 0)),
            pl.BlockSpec((NC, NS, BLK), lambda i: (0, 0, i)),
            pl.BlockSpec((BLK, D), lambda i: (i, 0)),
            pl.BlockSpec((D, D), lambda i: (0, 0)),
            pl.BlockSpec((D, D), lambda i: (0, 0)),
        ],
        out_specs=pl.BlockSpec((BLK, D), lambda i: (i, 0)),
        out_shape=jax.ShapeDtypeStruct((NPAD, D), jnp.float32),
    )(part, deg, x_pad, W, B)


def kernel(x, edge_index, W, B):
    src = edge_index[0].astype(jnp.int32)
    dst = edge_index[1].astype(jnp.int32)
    pad = E_PAD - N_EDGES
    src_p = jnp.concatenate([src, jnp.zeros((pad,), jnp.int32)])
    dst_p = jnp.concatenate([dst, jnp.full((pad,), N_NODES, jnp.int32)])
    src3 = src_p.reshape(TOTC, CHUNK)
    dst3 = dst_p.reshape(TOTC, CHUNK)
    ei3 = jnp.stack([src3, dst3], axis=1)

    part, deg = _sc_aggregate(x, ei3)
    x_pad = jnp.pad(x, ((0, NPAD - N_NODES), (0, 0)))
    out = _tc_combine(part, deg, x_pad, W, B)
    return out[:N_NODES]


# speedup vs baseline: 1.4734x; 1.3853x over previous
import functools

import jax
import jax.numpy as jnp
from jax import lax
from jax.experimental import pallas as pl
from jax.experimental.pallas import tpu as pltpu
from jax.experimental.pallas import tpu_sc as plsc

N_NODES = 10000
N_EDGES = 320000
D = 128

NC = 2
NS = 16
NW = NC * NS
CHUNK = 128
NCHUNK = -(-N_EDGES // (NW * CHUNK))
EPW = NCHUNK * CHUNK
NPAD = 10240
RPT = NPAD // NS
VPC = CHUNK // 16


def _sc_aggregate(x, src3, dst3):
    mesh = plsc.VectorSubcoreMesh(
        core_axis_name="c", subcore_axis_name="s", num_cores=NC, num_subcores=NS
    )

    @functools.partial(
        pl.kernel,
        out_type=[
            jax.ShapeDtypeStruct((NC, NPAD, D), jnp.float32),
            jax.ShapeDtypeStruct((NC, NS, NPAD), jnp.float32),
        ],
        mesh=mesh,
        compiler_params=pltpu.CompilerParams(needs_layout_passes=False),
        scratch_types=[
            pltpu.VMEM((NCHUNK, CHUNK), jnp.int32),
            pltpu.VMEM((NCHUNK, CHUNK), jnp.int32),
            pltpu.VMEM((CHUNK, D), jnp.float32),
            pltpu.VMEM((NPAD,), jnp.float32),
            pltpu.VMEM_SHARED((NPAD, D), jnp.float32),
        ],
    )
    def agg(x_hbm, src_hbm, dst_hbm, part_hbm, hist_hbm,
            src_v, dst_v, rows_v, hist_v, accum_sh):
        c = lax.axis_index("c")
        s = lax.axis_index("s")
        wid = c * NS + s

        zeros16 = jnp.zeros((16,), jnp.float32)

        def z_hist(i, _):
            hist_v[pl.ds(i * 16, 16)] = zeros16
            return 0
        lax.fori_loop(0, NPAD // 16, z_hist, 0)

        def z_rows(i, _):
            for v in range(D // 16):
                rows_v[i, pl.ds(v * 16, 16)] = zeros16
            return 0
        lax.fori_loop(0, CHUNK, z_rows, 0)

        pltpu.sync_copy(src_hbm.at[wid], src_v)
        pltpu.sync_copy(dst_hbm.at[wid], dst_v)

        for k in range(RPT // CHUNK):
            pltpu.sync_copy(rows_v, accum_sh.at[pl.ds(s * RPT + k * CHUNK, CHUNK)])
        plsc.subcore_barrier()

        def body(j, _):
            pltpu.sync_copy(x_hbm.at[src_v.at[j]], rows_v)
            pltpu.sync_copy(rows_v, accum_sh.at[dst_v.at[j]], add=True)
            for v in range(VPC):
                dvec = dst_v[j, pl.ds(v * 16, 16)]
                occ, last = plsc.scan_count(dvec)
                cur = plsc.load_gather(hist_v, [dvec], mask=last)
                plsc.store_scatter(hist_v, [dvec],
                                   cur + occ.astype(jnp.float32), mask=last)
            return 0

        lax.fori_loop(0, NCHUNK, body, 0)
        plsc.subcore_barrier()

        pltpu.sync_copy(accum_sh.at[pl.ds(s * RPT, RPT)],
                        part_hbm.at[c, pl.ds(s * RPT, RPT)])
        pltpu.sync_copy(hist_v, hist_hbm.at[c, s])

    return agg(x, src3, dst3)


def _tc_combine_body(p_ref, h_ref, x_ref, w_ref, b_ref, o_ref):
    agg = p_ref[0] + p_ref[1]
    deg = jnp.sum(h_ref[...], axis=1)
    deg = jnp.maximum(deg, 1.0)
    agg = agg / deg[:, None]
    o_ref[...] = (
        lax.dot_general(agg, w_ref[...], (((1,), (1,)), ((), ())),
                        preferred_element_type=jnp.float32)
        + lax.dot_general(x_ref[...], b_ref[...], (((1,), (1,)), ((), ())),
                          preferred_element_type=jnp.float32)
    )


def _tc_combine(part, histT, x, W, B):
    BLK = 1000
    grid = (N_NODES // BLK,)
    return pl.pallas_call(
        _tc_combine_body,
        grid=grid,
        in_specs=[
            pl.BlockSpec((NC, BLK, D), lambda i: (0, i, 0)),
            pl.BlockSpec((BLK, NW), lambda i: (i, 0)),
            pl.BlockSpec((BLK, D), lambda i: (i, 0)),
            pl.BlockSpec((D, D), lambda i: (0, 0)),
            pl.BlockSpec((D, D), lambda i: (0, 0)),
        ],
        out_specs=pl.BlockSpec((BLK, D), lambda i: (i, 0)),
        out_shape=jax.ShapeDtypeStruct((N_NODES, D), jnp.float32),
    )(part, histT, x, W, B)


def kernel(x, edge_index, W, B):
    src = edge_index[0].astype(jnp.int32)
    dst = edge_index[1].astype(jnp.int32)
    pad = NW * EPW - N_EDGES
    src_p = jnp.concatenate([src, jnp.zeros((pad,), jnp.int32)])
    dst_p = jnp.concatenate([dst, jnp.full((pad,), N_NODES, jnp.int32)])
    src3 = src_p.reshape(NW, NCHUNK, CHUNK)
    dst3 = dst_p.reshape(NW, NCHUNK, CHUNK)

    part, hist = _sc_aggregate(x, src3, dst3)
    histT = hist.reshape(NW, NPAD).T
    return _tc_combine(part, histT, x, W, B)
